# Initial kernel scaffold; baseline (speedup 1.0000x reference)
import jax
import jax.numpy as jnp
from jax.experimental import pallas as pl


def kernel(samples, clique_weights, rows):
    raise NotImplementedError("write your pallas kernel here")


# speedup vs baseline: 17.7039x; 17.7039x over previous
import functools

import jax
import jax.numpy as jnp
from jax import lax
from jax.experimental import pallas as pl
from jax.experimental.pallas import tpu as pltpu
from jax.experimental.pallas import tpu_sc as plsc

N_VARS = 16
N_CLIQUES = 15
K = 2 * N_VARS
L = 16
NC, NS = 2, 16
NW = NC * NS
OFF = 1

_UNIVERSE4 = (
    (1.0, 0.0, 1.0, 0.0),
    (1.0, 0.0, 0.0, 1.0),
    (0.0, 1.0, 1.0, 0.0),
    (0.0, 1.0, 0.0, 1.0),
)


def _splat(ref, flat_idx):
    assert flat_idx > 0
    return plsc.load_gather(ref, [jnp.full((L,), flat_idx, jnp.int32)])


def _make_sc_call(B):
    bpw = B // NW
    mesh = plsc.VectorSubcoreMesh(
        core_axis_name="c", subcore_axis_name="s", num_cores=NC)

    @functools.partial(
        pl.kernel,
        out_type=jax.ShapeDtypeStruct((B,), jnp.float32),
        mesh=mesh,
        compiler_params=pltpu.CompilerParams(
            needs_layout_passes=False, use_tc_tiling_on_sc=False),
        scratch_types=[
            pltpu.VMEM((bpw, K), jnp.float32),
            pltpu.VMEM((64,), jnp.float32),
            pltpu.VMEM((64,), jnp.int32),
            pltpu.VMEM((bpw,), jnp.float32),
        ],
    )
    def run(samples_hbm, w_hbm, r_hbm, out_hbm, s_v, w_v, r_v, o_v):
        wid = lax.axis_index("s") * NC + lax.axis_index("c")
        base = wid * bpw
        pltpu.sync_copy(w_hbm, w_v)
        pltpu.sync_copy(r_hbm, r_v)
        pltpu.sync_copy(samples_hbm.at[pl.ds(base, bpw)], s_v)

        a0 = jnp.full((L,), 1.0, jnp.float32)
        a1 = jnp.full((L,), 1.0, jnp.float32)
        for c in range(N_CLIQUES):
            w0 = _splat(w_v, OFF + 4 * c + 0)
            w1 = _splat(w_v, OFF + 4 * c + 1)
            w2 = _splat(w_v, OFF + 4 * c + 2)
            w3 = _splat(w_v, OFF + 4 * c + 3)
            a0, a1 = a0 * w0 + a1 * w2, a0 * w1 + a1 * w3
        inv_z = 1.0 / (a0 + a1)

        last = OFF + 4 * (N_CLIQUES - 1)
        rcol = [_splat(r_v, last + j) for j in range(4)]
        w14 = [_splat(w_v, last + w) for w in range(4)]
        wscaled = [w14[w] * inv_z for w in range(4)]

        def group(g, _):
            rowv = lax.iota(jnp.int32, L) + g * L
            f = [plsc.load_gather(s_v, [rowv, rcol[j]]) for j in range(4)]
            acc = None
            for w in range(4):
                u = _UNIVERSE4[w]
                m = ((f[0] == u[0]) & (f[1] == u[1])
                     & (f[2] == u[2]) & (f[3] == u[3]))
                term = m.astype(jnp.float32) * wscaled[w]
                acc = term if acc is None else acc + term
            o_v[pl.ds(g * L, L)] = acc
            return _

        lax.fori_loop(0, bpw // L, group, 0)
        pltpu.sync_copy(o_v, out_hbm.at[pl.ds(base, bpw)])

    return run


def kernel(samples, clique_weights, rows):
    B = samples.shape[0]
    pad = 64 - 4 * N_CLIQUES - OFF
    w_flat = jnp.pad(clique_weights.reshape(-1).astype(jnp.float32),
                     (OFF, pad))
    r_flat = jnp.pad(rows.reshape(-1).astype(jnp.int32), (OFF, pad))
    out = _make_sc_call(B)(samples.astype(jnp.float32), w_flat, r_flat)
    return out.astype(samples.dtype)


# speedup vs baseline: 18.5685x; 1.0488x over previous
import functools

import jax
import jax.numpy as jnp
from jax import lax
from jax.experimental import pallas as pl
from jax.experimental.pallas import tpu as pltpu
from jax.experimental.pallas import tpu_sc as plsc

N_VARS = 16
N_CLIQUES = 15
K = 2 * N_VARS
L = 16
NC, NS = 2, 16
NW = NC * NS
OFF = 1


def _splat(ref, flat_idx):
    assert flat_idx > 0
    return plsc.load_gather(ref, [jnp.full((L,), flat_idx, jnp.int32)])


def _make_sc_call(B):
    bpw = B // NW
    mesh = plsc.VectorSubcoreMesh(
        core_axis_name="c", subcore_axis_name="s", num_cores=NC)

    @functools.partial(
        pl.kernel,
        out_type=jax.ShapeDtypeStruct((B,), jnp.float32),
        mesh=mesh,
        compiler_params=pltpu.CompilerParams(
            needs_layout_passes=False, use_tc_tiling_on_sc=False),
        scratch_types=[
            pltpu.VMEM((bpw, K), jnp.float32),
            pltpu.VMEM((64,), jnp.float32),
            pltpu.VMEM((64,), jnp.int32),
            pltpu.VMEM((bpw,), jnp.float32),
            pltpu.SemaphoreType.DMA,
            pltpu.SemaphoreType.DMA,
        ],
    )
    def run(samples_hbm, w_hbm, r_hbm, out_hbm,
            s_v, w_v, r_v, o_v, sem_t, sem_s):
        wid = lax.axis_index("s") * NC + lax.axis_index("c")
        base = wid * bpw
        cp_s = pltpu.make_async_copy(
            samples_hbm.at[pl.ds(base, bpw)], s_v, sem_s)
        cp_s.start()
        cp_w = pltpu.make_async_copy(w_hbm, w_v, sem_t)
        cp_w.start()
        cp_r = pltpu.make_async_copy(r_hbm, r_v, sem_t)
        cp_r.start()
        cp_w.wait()
        cp_r.wait()

        offv = jnp.full((L,), OFF, jnp.int32)

        def dp_step(c, a):
            a0, a1 = a
            i0 = offv + 4 * c
            w0 = plsc.load_gather(w_v, [i0])
            w1 = plsc.load_gather(w_v, [i0 + 1])
            w2 = plsc.load_gather(w_v, [i0 + 2])
            w3 = plsc.load_gather(w_v, [i0 + 3])
            return (a0 * w0 + a1 * w2, a0 * w1 + a1 * w3)

        ones = jnp.full((L,), 1.0, jnp.float32)
        a0, a1 = lax.fori_loop(0, N_CLIQUES, dp_step, (ones, ones))
        inv_z = 1.0 / (a0 + a1)

        last = OFF + 4 * (N_CLIQUES - 1)
        rc1 = _splat(r_v, last + 1)
        rc3 = _splat(r_v, last + 3)
        cp_s.wait()

        @plsc.parallel_loop(0, bpw // L, unroll=4)
        def _group(g):
            rowv = lax.iota(jnp.int32, L) + g * L
            f1 = plsc.load_gather(s_v, [rowv, rc1])
            f3 = plsc.load_gather(s_v, [rowv, rc3])
            widx = (f1 + f1 + f3).astype(jnp.int32) + last
            o_v[pl.ds(g * L, L)] = plsc.load_gather(w_v, [widx]) * inv_z

        pltpu.sync_copy(o_v, out_hbm.at[pl.ds(base, bpw)])

    return run


def kernel(samples, clique_weights, rows):
    B = samples.shape[0]
    pad = 64 - 4 * N_CLIQUES - OFF
    w_flat = jnp.pad(clique_weights.reshape(-1).astype(jnp.float32),
                     (OFF, pad))
    r_flat = jnp.pad(rows.reshape(-1).astype(jnp.int32), (OFF, pad))
    out = _make_sc_call(B)(samples.astype(jnp.float32), w_flat, r_flat)
    return out.astype(samples.dtype)


# speedup vs baseline: 20.6941x; 1.1145x over previous
import functools

import jax
import jax.numpy as jnp
from jax import lax
from jax.experimental import pallas as pl
from jax.experimental.pallas import tpu as pltpu
from jax.experimental.pallas import tpu_sc as plsc

N_VARS = 16
N_CLIQUES = 15
K = 2 * N_VARS
L = 16
NC, NS = 2, 16
NW = NC * NS
OFF = 1


def _splat(ref, flat_idx):
    assert flat_idx > 0
    return plsc.load_gather(ref, [jnp.full((L,), flat_idx, jnp.int32)])


def _make_sc_call(B):
    bpw = B // NW
    mesh = plsc.VectorSubcoreMesh(
        core_axis_name="c", subcore_axis_name="s", num_cores=NC)

    @functools.partial(
        pl.kernel,
        out_type=jax.ShapeDtypeStruct((B,), jnp.float32),
        mesh=mesh,
        compiler_params=pltpu.CompilerParams(
            needs_layout_passes=False, use_tc_tiling_on_sc=True),
        scratch_types=[
            pltpu.VMEM((bpw, K), jnp.float32),
            pltpu.VMEM((64,), jnp.float32),
            pltpu.VMEM((64,), jnp.int32),
            pltpu.VMEM((bpw,), jnp.float32),
            pltpu.SemaphoreType.DMA,
            pltpu.SemaphoreType.DMA,
        ],
    )
    def run(samples_hbm, w_hbm, r_hbm, out_hbm,
            s_v, w_v, r_v, o_v, sem_t, sem_s):
        wid = lax.axis_index("s") * NC + lax.axis_index("c")
        base = wid * bpw
        cp_s = pltpu.make_async_copy(
            samples_hbm.at[pl.ds(base, bpw)], s_v, sem_s)
        cp_s.start()
        cp_w = pltpu.make_async_copy(w_hbm, w_v, sem_t)
        cp_w.start()
        cp_r = pltpu.make_async_copy(r_hbm, r_v, sem_t)
        cp_r.start()
        cp_w.wait()
        cp_r.wait()

        offv = jnp.full((L,), OFF, jnp.int32)

        def dp_step(c, a):
            a0, a1 = a
            i0 = offv + 4 * c
            w0 = plsc.load_gather(w_v, [i0])
            w1 = plsc.load_gather(w_v, [i0 + 1])
            w2 = plsc.load_gather(w_v, [i0 + 2])
            w3 = plsc.load_gather(w_v, [i0 + 3])
            return (a0 * w0 + a1 * w2, a0 * w1 + a1 * w3)

        ones = jnp.full((L,), 1.0, jnp.float32)
        a0, a1 = lax.fori_loop(0, N_CLIQUES, dp_step, (ones, ones))
        inv_z = 1.0 / (a0 + a1)

        last = OFF + 4 * (N_CLIQUES - 1)
        rc1 = _splat(r_v, last + 1)
        rc3 = _splat(r_v, last + 3)
        cp_s.wait()

        @plsc.parallel_loop(0, bpw // L, unroll=4)
        def _group(g):
            rowv = lax.iota(jnp.int32, L) + g * L
            f1 = plsc.load_gather(s_v, [rowv, rc1])
            f3 = plsc.load_gather(s_v, [rowv, rc3])
            widx = (f1 + f1 + f3).astype(jnp.int32) + last
            o_v[pl.ds(g * L, L)] = plsc.load_gather(w_v, [widx]) * inv_z

        pltpu.sync_copy(o_v, out_hbm.at[pl.ds(base, bpw)])

    return run


def kernel(samples, clique_weights, rows):
    B = samples.shape[0]
    pad = 64 - 4 * N_CLIQUES - OFF
    w_flat = jnp.pad(clique_weights.reshape(-1).astype(jnp.float32),
                     (OFF, pad))
    r_flat = jnp.pad(rows.reshape(-1).astype(jnp.int32), (OFF, pad))
    out = _make_sc_call(B)(samples.astype(jnp.float32), w_flat, r_flat)
    return out.astype(samples.dtype)
